# baseline (speedup 1.0000x reference)
import jax
import jax.numpy as jnp
from jax.experimental import pallas as pl


def kernel(nodes, mask, We, be, Wn, bn):
    mixed = -2.0 * jnp.einsum('bnf,bmf->bnm', nodes, nodes)
    xs = jnp.sum(nodes * nodes, axis=2)[:, :, None]
    ys = jnp.sum(nodes * nodes, axis=2)[:, None, :]
    dist = jnp.abs(mixed + xs + ys)
    _, idx = jax.lax.top_k(-dist, 16 + 1)
    idx = idx[..., 1:]
    neigh = jnp.take_along_axis(nodes[:, None, :, :], idx[..., None], axis=2)
    recv = jnp.broadcast_to(nodes[:, :, None, :], neigh.shape)
    edges = jnp.concatenate([recv, neigh], axis=-1)
    e = jax.nn.relu(edges @ We + be)
    pooled = jnp.mean(e, axis=2)
    h = jnp.concatenate([nodes, pooled], axis=-1)
    out = jax.nn.relu(h @ Wn + bn)
    return out * mask[..., None]


# speedup vs baseline: 10.9764x; 10.9764x over previous
import functools

import jax
import jax.numpy as jnp
from jax import lax
from jax.experimental import pallas as pl
from jax.experimental.pallas import tpu as pltpu
from jax.experimental.pallas import tpu_sc as plsc

B, N, F, H, K = 8, 2048, 64, 128, 16
BN = B * N
T = 256
NT = N // T
NC, NS = 2, 16
NW = NC * NS
NPW = BN // NW
CH = 32
NCH = NPW // CH
TC_T = 1024


def _knn_body(x_ref, xa_ref, wer_ref, wes_ref, be_ref, idx_ref, r_ref, s_ref):
    b = pl.program_id(0)
    t = pl.program_id(1)
    x = x_ref[0]
    xa = xa_ref[0]
    mm = lax.dot_general(x, xa, (((1,), (1,)), ((), ())),
                         preferred_element_type=jnp.float32)
    xs = jnp.sum(x * x, axis=1, keepdims=True)
    ys = jnp.sum(xa * xa, axis=1)[None, :]
    dist = jnp.abs(-2.0 * mm + xs + ys)
    cols = lax.broadcasted_iota(jnp.int32, (T, N), 1)
    rows = lax.broadcasted_iota(jnp.int32, (T, N), 0) + t * T
    dist = jnp.where(cols == rows, jnp.inf, dist)
    base = b * N
    picks = []
    for _ in range(K):
        m = jnp.min(dist, axis=1, keepdims=True)
        sel = jnp.where(dist == m, cols, N)
        amin = jnp.min(sel, axis=1, keepdims=True)
        picks.append(amin + base)
        dist = jnp.where(cols == amin, jnp.inf, dist)
    idx_ref[...] = jnp.concatenate(picks, axis=1)
    r_ref[...] = lax.dot_general(x, wer_ref[...], (((1,), (0,)), ((), ())),
                                 preferred_element_type=jnp.float32) + be_ref[...]
    s_ref[...] = lax.dot_general(x, wes_ref[...], (((1,), (0,)), ((), ())),
                                 preferred_element_type=jnp.float32)


def _knn_stage(nodes, We_r, We_s, be2):
    return pl.pallas_call(
        _knn_body,
        grid=(B, NT),
        in_specs=[
            pl.BlockSpec((1, T, F), lambda b, t: (b, t, 0)),
            pl.BlockSpec((1, N, F), lambda b, t: (b, 0, 0)),
            pl.BlockSpec((F, H), lambda b, t: (0, 0)),
            pl.BlockSpec((F, H), lambda b, t: (0, 0)),
            pl.BlockSpec((1, H), lambda b, t: (0, 0)),
        ],
        out_specs=[
            pl.BlockSpec((T, K), lambda b, t: (b * NT + t, 0)),
            pl.BlockSpec((T, H), lambda b, t: (b * NT + t, 0)),
            pl.BlockSpec((T, H), lambda b, t: (b * NT + t, 0)),
        ],
        out_shape=[
            jax.ShapeDtypeStruct((BN, K), jnp.int32),
            jax.ShapeDtypeStruct((BN, H), jnp.float32),
            jax.ShapeDtypeStruct((BN, H), jnp.float32),
        ],
    )(nodes, nodes, We_r, We_s, be2)


def _pool_body(idx_hbm, s_hbm, r_hbm, out_hbm, idx_c, rows_v, r_v, acc_v, sem):
    wid = lax.axis_index("s") * NC + lax.axis_index("c")
    base = wid * NPW
    pltpu.sync_copy(
        idx_hbm.at[pl.ds(pl.multiple_of(base * K // 128, 8), NPW * K // 128)], idx_c)

    def chunk(c, carry):
        nbase = pl.multiple_of(base + c * CH, 8)
        pltpu.sync_copy(r_hbm.at[pl.ds(nbase, CH)], r_v)
        for j in range(CH * K // 128):
            pltpu.async_copy(s_hbm.at[idx_c.at[c * (CH * K // 128) + j]],
                             rows_v.at[pl.ds(j * 128, 128)], sem).wait()

        def node(n, carry2):
            acc = [jnp.zeros((16,), jnp.float32) for _ in range(H // 16)]
            rv = [r_v[n, pl.ds(16 * v, 16)] for v in range(H // 16)]
            for k in range(K):
                for v in range(H // 16):
                    sv = rows_v[n * K + k, pl.ds(16 * v, 16)]
                    acc[v] = acc[v] + jnp.maximum(sv + rv[v], 0.0)
            for v in range(H // 16):
                acc_v[n, pl.ds(16 * v, 16)] = acc[v] * (1.0 / K)
            return carry2

        lax.fori_loop(0, CH, node, 0)
        pltpu.sync_copy(acc_v, out_hbm.at[pl.ds(nbase, CH)])
        return carry

    lax.fori_loop(0, NCH, chunk, 0)


def _pool_stage(idx2d, s, r):
    mesh = plsc.VectorSubcoreMesh(core_axis_name="c", subcore_axis_name="s")
    return pl.kernel(
        _pool_body,
        out_type=jax.ShapeDtypeStruct((BN, H), jnp.float32),
        mesh=mesh,
        scratch_types=[
            pltpu.VMEM((NPW * K // 128, 128), jnp.int32),
            pltpu.VMEM((CH * K, H), jnp.float32),
            pltpu.VMEM((CH, H), jnp.float32),
            pltpu.VMEM((CH, H), jnp.float32),
            pltpu.SemaphoreType.DMA,
        ],
    )(idx2d, s, r)


def _node_body(x_ref, p_ref, wnf_ref, wnp_ref, bn_ref, m_ref, o_ref):
    h = (lax.dot_general(x_ref[...], wnf_ref[...], (((1,), (0,)), ((), ())),
                         preferred_element_type=jnp.float32)
         + lax.dot_general(p_ref[...], wnp_ref[...], (((1,), (0,)), ((), ())),
                           preferred_element_type=jnp.float32)
         + bn_ref[...])
    o_ref[...] = jnp.maximum(h, 0.0) * m_ref[...]


def _node_stage(nodes2, pooled, Wn_f, Wn_p, bn2, mask2):
    return pl.pallas_call(
        _node_body,
        grid=(BN // TC_T,),
        in_specs=[
            pl.BlockSpec((TC_T, F), lambda i: (i, 0)),
            pl.BlockSpec((TC_T, H), lambda i: (i, 0)),
            pl.BlockSpec((F, H), lambda i: (0, 0)),
            pl.BlockSpec((H, H), lambda i: (0, 0)),
            pl.BlockSpec((1, H), lambda i: (0, 0)),
            pl.BlockSpec((TC_T, 1), lambda i: (i, 0)),
        ],
        out_specs=pl.BlockSpec((TC_T, H), lambda i: (i, 0)),
        out_shape=jax.ShapeDtypeStruct((BN, H), jnp.float32),
    )(nodes2, pooled, Wn_f, Wn_p, bn2, mask2)


def kernel(nodes, mask, We, be, Wn, bn):
    We_r = We[:F]
    We_s = We[F:]
    Wn_f = Wn[:F]
    Wn_p = Wn[F:]
    be2 = be.reshape(1, H)
    bn2 = bn.reshape(1, H)
    idx, r, s = _knn_stage(nodes, We_r, We_s, be2)
    idx2d = idx.reshape(BN * K // 128, 128)
    pooled = _pool_stage(idx2d, s, r)
    nodes2 = nodes.reshape(BN, F)
    mask2 = mask.reshape(BN, 1)
    out = _node_stage(nodes2, pooled, Wn_f, Wn_p, bn2, mask2)
    return out.reshape(B, N, H)
